# Initial kernel scaffold; baseline (speedup 1.0000x reference)
import functools

import jax
import jax.numpy as jnp
from jax import lax
from jax.experimental import pallas as pl
from jax.experimental.pallas import tpu as pltpu
from jax.experimental.pallas import tpu_sc as plsc

_N = 10000
_E = 320000
_G = 64
_CIN = 128

_NC = 2
_NS = 16
_CH = 128
_CPT = 80
_EPAD = _NC * _NS * _CPT * _CH
_NPAD = _N + 8
_RPT = _N // _NS
_DW = 16

_sc_mesh = plsc.VectorSubcoreMesh(core_axis_name="c", subcore_axis_name="s")


def _deg_body(dstb, zeros, ones, out, ones_v, idx_d, sem, acc):
    c = lax.axis_index("c")
    s = lax.axis_index("s")
    wid = c * _NS + s
    pltpu.sync_copy(zeros, acc.at[pl.ds(s * _RPT, _RPT)])
    pltpu.sync_copy(ones, ones_v)
    pltpu.sync_copy(dstb.at[pl.ds(wid * _CPT, _CPT)], idx_d)
    plsc.subcore_barrier()

    def chunk(j, carry):
        pltpu.sync_copy(ones_v, acc.at[idx_d.at[j]], add=True)
        return carry

    lax.fori_loop(0, _CPT, chunk, 0)
    plsc.subcore_barrier()
    pltpu.sync_copy(acc.at[pl.ds(s * _RPT, _RPT)],
                    out.at[c, pl.ds(s * _RPT, _RPT)])


_deg_call = pl.kernel(
    _deg_body,
    out_type=jax.ShapeDtypeStruct((_NC, _N, _DW), jnp.float32),
    mesh=_sc_mesh,
    scratch_types=[
        pltpu.VMEM((_CH, _DW), jnp.float32),
        pltpu.VMEM((_CPT, _CH), jnp.int32),
        pltpu.SemaphoreType.DMA,
        pltpu.VMEM_SHARED((_NPAD, _DW), jnp.float32),
    ],
)


def _agg_body(table, srcb, dstb, zeros, out, idx_s, idx_d, rows, sem, acc):
    c = lax.axis_index("c")
    s = lax.axis_index("s")
    wid = c * _NS + s
    pltpu.sync_copy(zeros, acc.at[pl.ds(s * _RPT, _RPT)])
    pltpu.sync_copy(srcb.at[pl.ds(wid * _CPT, _CPT)], idx_s)
    pltpu.sync_copy(dstb.at[pl.ds(wid * _CPT, _CPT)], idx_d)
    plsc.subcore_barrier()

    def chunk(j, carry):
        pltpu.async_copy(table.at[idx_s.at[j]], rows, sem).wait()
        pltpu.sync_copy(rows, acc.at[idx_d.at[j]], add=True)
        return carry

    lax.fori_loop(0, _CPT, chunk, 0)
    plsc.subcore_barrier()
    pltpu.sync_copy(acc.at[pl.ds(s * _RPT, _RPT)],
                    out.at[c, pl.ds(s * _RPT, _RPT)])


_agg_call = pl.kernel(
    _agg_body,
    out_type=jax.ShapeDtypeStruct((_NC, _N, _CIN), jnp.float32),
    mesh=_sc_mesh,
    scratch_types=[
        pltpu.VMEM((_CPT, _CH), jnp.int32),
        pltpu.VMEM((_CPT, _CH), jnp.int32),
        pltpu.VMEM((_CH, _CIN), jnp.float32),
        pltpu.SemaphoreType.DMA,
        pltpu.VMEM_SHARED((_NPAD, _CIN), jnp.float32),
    ],
)


def _prep_body(x_ref, d0_ref, d1_ref, xs_ref, dinv_ref):
    deg = d0_ref[:, 0:1] + d1_ref[:, 0:1] + 1.0
    dinv = lax.rsqrt(deg)
    dinv_ref[...] = dinv
    xs_ref[...] = x_ref[...] * dinv


def _layer1_body(a0_ref, a1_ref, xs_ref, dinv_ref, w_ref, b_ref,
                 ha_ref, hb_ref):
    dinv = dinv_ref[...]
    y = (a0_ref[...] + a1_ref[...] + xs_ref[...]) * dinv
    h = jnp.maximum(
        jnp.dot(y, w_ref[...], preferred_element_type=jnp.float32)
        + b_ref[...], 0.0) * dinv
    ha_ref[...] = h[:, :_CIN]
    hb_ref[...] = h[:, _CIN:]


def _layer2_pool_body(gl_ref, a2a0, a2a1, a2b0, a2b1, hsa, hsb,
                      dinv_ref, bat_ref, w_ref, b_ref, pool_ref):
    blk = pl.program_id(0)

    @pl.when(blk == 0)
    def _init():
        pool_ref[...] = jnp.full_like(pool_ref[...], -jnp.inf)

    dinv = dinv_ref[...]
    ya = (a2a0[...] + a2a1[...] + hsa[...]) * dinv
    yb = (a2b0[...] + a2b1[...] + hsb[...]) * dinv
    y = jnp.concatenate([ya, yb], axis=1)
    h = jnp.maximum(
        jnp.dot(y, w_ref[...], preferred_element_type=jnp.float32)
        + b_ref[...], 0.0)
    bat = bat_ref[...]

    def upd(g, carry):
        m = bat == g
        mx = jnp.max(jnp.where(m, h, -jnp.inf), axis=0, keepdims=True)
        pool_ref[pl.ds(g, 1), :] = jnp.maximum(pool_ref[pl.ds(g, 1), :], mx)
        return carry

    lax.fori_loop(gl_ref[0, blk], gl_ref[1, blk] + 1, upd, 0)


def _head_body(p_ref, m1w, m1b, g1, be1, m2w, m2b, g2, be2, lw, lb, o_ref):
    p = p_ref[...]
    p = jnp.where(jnp.isfinite(p), p, 0.0)
    eps = 1e-5

    def bn(z, g, b):
        m = jnp.mean(z, axis=0, keepdims=True)
        v = jnp.mean((z - m) ** 2, axis=0, keepdims=True)
        return g[...] * (z - m) / jnp.sqrt(v + eps) + b[...]

    z = jnp.maximum(jnp.dot(p, m1w[...], preferred_element_type=jnp.float32)
                    + m1b[...], 0.0)
    z = bn(z, g1, be1)
    z = jnp.maximum(jnp.dot(z, m2w[...], preferred_element_type=jnp.float32)
                    + m2b[...], 0.0)
    z = bn(z, g2, be2)
    z = jnp.dot(z, lw[...], preferred_element_type=jnp.float32) + lb[...]
    zmax = jnp.max(z, axis=1, keepdims=True)
    zs = z - zmax
    o_ref[...] = zs - jnp.log(jnp.sum(jnp.exp(zs), axis=1, keepdims=True))


def _row_spec(rb, cols):
    return pl.BlockSpec((rb, cols), lambda i: (i, 0))


def _full_spec(shape):
    return pl.BlockSpec(shape, lambda i: tuple(0 for _ in shape))


def kernel(x, edge_index, batch, W1, b1, W2, b2, M1W, M1b, BN1g, BN1b,
           M2W, M2b, BN2g, BN2b, LW, Lb):
    f32 = jnp.float32
    src = edge_index[0].astype(jnp.int32)
    dst = edge_index[1].astype(jnp.int32)
    pad = _EPAD - _E
    srcb = jnp.concatenate([src, jnp.zeros((pad,), jnp.int32)])
    srcb = srcb.reshape(_EPAD // _CH, _CH)
    dstb = jnp.concatenate([dst, jnp.full((pad,), _N, jnp.int32)])
    dstb = dstb.reshape(_EPAD // _CH, _CH)

    zeros_deg = jnp.zeros((_RPT, _DW), f32)
    ones_deg = jnp.ones((_CH, _DW), f32)
    zeros_agg = jnp.zeros((_RPT, _CIN), f32)

    degp = _deg_call(dstb, zeros_deg, ones_deg)

    rb = 1000
    grid = (_N // rb,)
    xs, dinv = pl.pallas_call(
        _prep_body,
        grid=grid,
        in_specs=[_row_spec(rb, _CIN), _row_spec(rb, _DW), _row_spec(rb, _DW)],
        out_specs=[_row_spec(rb, _CIN), _row_spec(rb, 1)],
        out_shape=[jax.ShapeDtypeStruct((_N, _CIN), f32),
                   jax.ShapeDtypeStruct((_N, 1), f32)],
    )(x, degp[0], degp[1])

    a1 = _agg_call(xs, srcb, dstb, zeros_agg)

    rd = 500
    gridd = (_N // rd,)
    hsa, hsb = pl.pallas_call(
        _layer1_body,
        grid=gridd,
        in_specs=[_row_spec(rd, _CIN), _row_spec(rd, _CIN),
                  _row_spec(rd, _CIN), _row_spec(rd, 1),
                  _full_spec((_CIN, 2 * _CIN)), _full_spec((1, 2 * _CIN))],
        out_specs=[_row_spec(rd, _CIN), _row_spec(rd, _CIN)],
        out_shape=[jax.ShapeDtypeStruct((_N, _CIN), f32),
                   jax.ShapeDtypeStruct((_N, _CIN), f32)],
    )(a1[0], a1[1], xs, dinv, W1, b1.reshape(1, -1))

    a2a = _agg_call(hsa, srcb, dstb, zeros_agg)
    a2b = _agg_call(hsb, srcb, dstb, zeros_agg)

    nblk = _N // rd
    bi = jnp.arange(nblk, dtype=jnp.int32)
    bat32 = batch.astype(jnp.int32)
    gl = jnp.stack([bat32[bi * rd], bat32[(bi + 1) * rd - 1]])
    pooled = pl.pallas_call(
        _layer2_pool_body,
        grid_spec=pltpu.PrefetchScalarGridSpec(
            num_scalar_prefetch=1,
            grid=gridd,
            in_specs=[_row_spec(rd, _CIN), _row_spec(rd, _CIN),
                      _row_spec(rd, _CIN), _row_spec(rd, _CIN),
                      _row_spec(rd, _CIN), _row_spec(rd, _CIN),
                      _row_spec(rd, 1), _row_spec(rd, 1),
                      _full_spec((2 * _CIN, 4 * _CIN)),
                      _full_spec((1, 4 * _CIN))],
            out_specs=_full_spec((_G, 4 * _CIN)),
        ),
        out_shape=jax.ShapeDtypeStruct((_G, 4 * _CIN), f32),
    )(gl, a2a[0], a2a[1], a2b[0], a2b[1], hsa, hsb, dinv,
      bat32.reshape(_N, 1), W2, b2.reshape(1, -1))

    out = pl.pallas_call(
        _head_body,
        grid=(1,),
        in_specs=[_full_spec((_G, 4 * _CIN)),
                  _full_spec((4 * _CIN, 32)), _full_spec((1, 32)),
                  _full_spec((1, 32)), _full_spec((1, 32)),
                  _full_spec((32, 64)), _full_spec((1, 64)),
                  _full_spec((1, 64)), _full_spec((1, 64)),
                  _full_spec((64, 40)), _full_spec((1, 40))],
        out_specs=_full_spec((_G, 40)),
        out_shape=jax.ShapeDtypeStruct((_G, 40), f32),
    )(pooled, M1W, M1b.reshape(1, -1), BN1g.reshape(1, -1),
      BN1b.reshape(1, -1), M2W, M2b.reshape(1, -1), BN2g.reshape(1, -1),
      BN2b.reshape(1, -1), LW, Lb.reshape(1, -1))
    return out


# speedup vs baseline: 8.0302x; 8.0302x over previous
import functools

import jax
import jax.numpy as jnp
from jax import lax
from jax.experimental import pallas as pl
from jax.experimental.pallas import tpu as pltpu
from jax.experimental.pallas import tpu_sc as plsc

_N = 10000
_E = 320000
_G = 64
_CIN = 128

_NC = 2
_NS = 16
_CH = 128
_CPT = 80
_EPAD = _NC * _NS * _CPT * _CH
_NPAD = 10240
_RPT = _NPAD // _NS
_DW = 128

@functools.lru_cache(maxsize=None)
def _sc_mesh():
    return plsc.VectorSubcoreMesh(core_axis_name="c", subcore_axis_name="s",
                                  num_cores=_NC, num_subcores=_NS)


def _deg_body(dstb, zeros, ones, out, ones_v, idx_d, sem, acc):
    c = lax.axis_index("c")
    s = lax.axis_index("s")
    wid = c * _NS + s
    pltpu.sync_copy(zeros, acc.at[pl.ds(s * _RPT, _RPT)])
    pltpu.sync_copy(ones, ones_v)
    pltpu.sync_copy(dstb.at[pl.ds(wid * _CPT, _CPT)], idx_d)
    plsc.subcore_barrier()

    def chunk(j, carry):
        pltpu.sync_copy(ones_v, acc.at[idx_d.at[j]], add=True)
        return carry

    lax.fori_loop(0, _CPT, chunk, 0)
    plsc.subcore_barrier()
    pltpu.sync_copy(acc.at[pl.ds(s * _RPT, _RPT)],
                    out.at[c, pl.ds(s * _RPT, _RPT)])


@functools.lru_cache(maxsize=None)
def _deg_call():
    return pl.kernel(
        _deg_body,
        out_type=jax.ShapeDtypeStruct((_NC, _NPAD, _DW), jnp.float32),
        mesh=_sc_mesh(),
        scratch_types=[
            pltpu.VMEM((_CH, _DW), jnp.float32),
            pltpu.VMEM((_CPT, _CH), jnp.int32),
            pltpu.SemaphoreType.DMA,
            pltpu.VMEM_SHARED((_NPAD, _DW), jnp.float32),
        ],
    )


def _agg_body(table, srcb, dstb, zeros, out, idx_s, idx_d, rows, sem, acc):
    c = lax.axis_index("c")
    s = lax.axis_index("s")
    wid = c * _NS + s
    pltpu.sync_copy(zeros, acc.at[pl.ds(s * _RPT, _RPT)])
    pltpu.sync_copy(srcb.at[pl.ds(wid * _CPT, _CPT)], idx_s)
    pltpu.sync_copy(dstb.at[pl.ds(wid * _CPT, _CPT)], idx_d)
    plsc.subcore_barrier()

    def chunk(j, carry):
        pltpu.async_copy(table.at[idx_s.at[j]], rows, sem).wait()
        pltpu.sync_copy(rows, acc.at[idx_d.at[j]], add=True)
        return carry

    lax.fori_loop(0, _CPT, chunk, 0)
    plsc.subcore_barrier()
    pltpu.sync_copy(acc.at[pl.ds(s * _RPT, _RPT)],
                    out.at[c, pl.ds(s * _RPT, _RPT)])


@functools.lru_cache(maxsize=None)
def _agg_call():
    return pl.kernel(
        _agg_body,
        out_type=jax.ShapeDtypeStruct((_NC, _NPAD, _CIN), jnp.float32),
        mesh=_sc_mesh(),
        scratch_types=[
            pltpu.VMEM((_CPT, _CH), jnp.int32),
            pltpu.VMEM((_CPT, _CH), jnp.int32),
            pltpu.VMEM((_CH, _CIN), jnp.float32),
            pltpu.SemaphoreType.DMA,
            pltpu.VMEM_SHARED((_NPAD, _CIN), jnp.float32),
        ],
    )


def _prep_body(x_ref, d0_ref, d1_ref, xs_ref, dinv_ref):
    deg = d0_ref[:, 0:1] + d1_ref[:, 0:1] + 1.0
    dinv = lax.rsqrt(deg)
    dinv_ref[...] = dinv
    xs_ref[...] = x_ref[...] * dinv


def _layer1_body(a0_ref, a1_ref, xs_ref, dinv_ref, w_ref, b_ref,
                 ha_ref, hb_ref):
    dinv = dinv_ref[...]
    y = (a0_ref[...] + a1_ref[...] + xs_ref[...]) * dinv
    h = jnp.maximum(
        jnp.dot(y, w_ref[...], preferred_element_type=jnp.float32)
        + b_ref[...], 0.0) * dinv
    ha_ref[...] = h[:, :_CIN]
    hb_ref[...] = h[:, _CIN:]


def _layer2_pool_body(gl_ref, a2a0, a2a1, a2b0, a2b1, hsa, hsb,
                      dinv_ref, bat_ref, w_ref, b_ref, pool_ref):
    blk = pl.program_id(0)

    @pl.when(blk == 0)
    def _init():
        pool_ref[...] = jnp.full_like(pool_ref[...], -jnp.inf)

    dinv = dinv_ref[...]
    ya = (a2a0[...] + a2a1[...] + hsa[...]) * dinv
    yb = (a2b0[...] + a2b1[...] + hsb[...]) * dinv
    y = jnp.concatenate([ya, yb], axis=1)
    h = jnp.maximum(
        jnp.dot(y, w_ref[...], preferred_element_type=jnp.float32)
        + b_ref[...], 0.0)
    bat = bat_ref[...]

    def upd(g, carry):
        m = bat == g
        mx = jnp.max(jnp.where(m, h, -jnp.inf), axis=0, keepdims=True)
        pool_ref[pl.ds(g, 1), :] = jnp.maximum(pool_ref[pl.ds(g, 1), :], mx)
        return carry

    lax.fori_loop(gl_ref[0, blk], gl_ref[1, blk] + 1, upd, 0)


def _head_body(p_ref, m1w, m1b, g1, be1, m2w, m2b, g2, be2, lw, lb, o_ref):
    p = p_ref[...]
    p = jnp.where(jnp.isfinite(p), p, 0.0)
    eps = 1e-5

    def bn(z, g, b):
        m = jnp.mean(z, axis=0, keepdims=True)
        v = jnp.mean((z - m) ** 2, axis=0, keepdims=True)
        return g[...] * (z - m) / jnp.sqrt(v + eps) + b[...]

    z = jnp.maximum(jnp.dot(p, m1w[...], preferred_element_type=jnp.float32)
                    + m1b[...], 0.0)
    z = bn(z, g1, be1)
    z = jnp.maximum(jnp.dot(z, m2w[...], preferred_element_type=jnp.float32)
                    + m2b[...], 0.0)
    z = bn(z, g2, be2)
    z = jnp.dot(z, lw[...], preferred_element_type=jnp.float32) + lb[...]
    zmax = jnp.max(z, axis=1, keepdims=True)
    zs = z - zmax
    o_ref[...] = zs - jnp.log(jnp.sum(jnp.exp(zs), axis=1, keepdims=True))


def _row_spec(rb, cols):
    return pl.BlockSpec((rb, cols), lambda i, *_: (i, 0))


def _full_spec(shape):
    return pl.BlockSpec(shape, lambda i, *_: tuple(0 for _ in shape))


def kernel(x, edge_index, batch, W1, b1, W2, b2, M1W, M1b, BN1g, BN1b,
           M2W, M2b, BN2g, BN2b, LW, Lb):
    f32 = jnp.float32
    src = edge_index[0].astype(jnp.int32)
    dst = edge_index[1].astype(jnp.int32)
    pad = _EPAD - _E
    srcb = jnp.concatenate([src, jnp.zeros((pad,), jnp.int32)])
    srcb = srcb.reshape(_EPAD // _CH, _CH)
    dstb = jnp.concatenate([dst, jnp.full((pad,), _N, jnp.int32)])
    dstb = dstb.reshape(_EPAD // _CH, _CH)

    zeros_deg = jnp.zeros((_RPT, _DW), f32)
    ones_deg = jnp.ones((_CH, _DW), f32)
    zeros_agg = zeros_deg

    degp = _deg_call()(dstb, zeros_deg, ones_deg)[:, :_N]

    rb = 1000
    grid = (_N // rb,)
    xs, dinv = pl.pallas_call(
        _prep_body,
        grid=grid,
        in_specs=[_row_spec(rb, _CIN), _row_spec(rb, _DW), _row_spec(rb, _DW)],
        out_specs=[_row_spec(rb, _CIN), _row_spec(rb, 1)],
        out_shape=[jax.ShapeDtypeStruct((_N, _CIN), f32),
                   jax.ShapeDtypeStruct((_N, 1), f32)],
    )(x, degp[0], degp[1])

    a1 = _agg_call()(xs, srcb, dstb, zeros_agg)[:, :_N]

    rd = 400
    gridd = (_N // rd,)
    hsa, hsb = pl.pallas_call(
        _layer1_body,
        grid=gridd,
        in_specs=[_row_spec(rd, _CIN), _row_spec(rd, _CIN),
                  _row_spec(rd, _CIN), _row_spec(rd, 1),
                  _full_spec((_CIN, 2 * _CIN)), _full_spec((1, 2 * _CIN))],
        out_specs=[_row_spec(rd, _CIN), _row_spec(rd, _CIN)],
        out_shape=[jax.ShapeDtypeStruct((_N, _CIN), f32),
                   jax.ShapeDtypeStruct((_N, _CIN), f32)],
    )(a1[0], a1[1], xs, dinv, W1, b1.reshape(1, -1))

    a2a = _agg_call()(hsa, srcb, dstb, zeros_agg)[:, :_N]
    a2b = _agg_call()(hsb, srcb, dstb, zeros_agg)[:, :_N]

    nblk = _N // rd
    bi = jnp.arange(nblk, dtype=jnp.int32)
    bat32 = batch.astype(jnp.int32)
    gl = jnp.stack([bat32[bi * rd], bat32[(bi + 1) * rd - 1]])
    pooled = pl.pallas_call(
        _layer2_pool_body,
        grid_spec=pltpu.PrefetchScalarGridSpec(
            num_scalar_prefetch=1,
            grid=gridd,
            in_specs=[_row_spec(rd, _CIN), _row_spec(rd, _CIN),
                      _row_spec(rd, _CIN), _row_spec(rd, _CIN),
                      _row_spec(rd, _CIN), _row_spec(rd, _CIN),
                      _row_spec(rd, 1), _row_spec(rd, 1),
                      _full_spec((2 * _CIN, 4 * _CIN)),
                      _full_spec((1, 4 * _CIN))],
            out_specs=_full_spec((_G, 4 * _CIN)),
        ),
        out_shape=jax.ShapeDtypeStruct((_G, 4 * _CIN), f32),
    )(gl, a2a[0], a2a[1], a2b[0], a2b[1], hsa, hsb, dinv,
      bat32.reshape(_N, 1), W2, b2.reshape(1, -1))

    out = pl.pallas_call(
        _head_body,
        grid=(1,),
        in_specs=[_full_spec((_G, 4 * _CIN)),
                  _full_spec((4 * _CIN, 32)), _full_spec((1, 32)),
                  _full_spec((1, 32)), _full_spec((1, 32)),
                  _full_spec((32, 64)), _full_spec((1, 64)),
                  _full_spec((1, 64)), _full_spec((1, 64)),
                  _full_spec((64, 40)), _full_spec((1, 40))],
        out_specs=_full_spec((_G, 40)),
        out_shape=jax.ShapeDtypeStruct((_G, 40), f32),
    )(pooled, M1W, M1b.reshape(1, -1), BN1g.reshape(1, -1),
      BN1b.reshape(1, -1), M2W, M2b.reshape(1, -1), BN2g.reshape(1, -1),
      BN2b.reshape(1, -1), LW, Lb.reshape(1, -1))
    return out


# speedup vs baseline: 9.9229x; 1.2357x over previous
import functools

import jax
import jax.numpy as jnp
from jax import lax
from jax.experimental import pallas as pl
from jax.experimental.pallas import tpu as pltpu
from jax.experimental.pallas import tpu_sc as plsc

_N = 10000
_E = 320000
_G = 64
_CIN = 128

_NC = 2
_NS = 16
_CH = 128
_CPT = 80
_CPT_A = 120
_CPT_B = 40
_CPT_MAX = max(_CPT_A, _CPT_B)
_EPAD = _NC * _NS * _CPT * _CH
_EROWS = _EPAD // _CH + _CPT_MAX
_NPAD = 10240
_RPT = _NPAD // _NS
_DW = 128

@functools.lru_cache(maxsize=None)
def _sc_mesh():
    return plsc.VectorSubcoreMesh(core_axis_name="c", subcore_axis_name="s",
                                  num_cores=_NC, num_subcores=_NS)


def _deg_body(dstb, zeros, ones, out, ones_v, idx_d, sem, acc):
    c = lax.axis_index("c")
    s = lax.axis_index("s")
    wid = c * _NS + s
    pltpu.sync_copy(zeros, acc.at[pl.ds(s * _RPT, _RPT)])
    pltpu.sync_copy(ones, ones_v)
    pltpu.sync_copy(dstb.at[pl.ds(wid * _CPT, _CPT)], idx_d)
    plsc.subcore_barrier()

    def chunk(j, carry):
        pltpu.sync_copy(ones_v, acc.at[idx_d.at[j]], add=True)
        return carry

    lax.fori_loop(0, _CPT, chunk, 0)
    plsc.subcore_barrier()
    pltpu.sync_copy(acc.at[pl.ds(s * _RPT, _RPT)],
                    out.at[c, pl.ds(s * _RPT, _RPT)])


@functools.lru_cache(maxsize=None)
def _deg_call():
    return pl.kernel(
        _deg_body,
        out_type=jax.ShapeDtypeStruct((_NC, _NPAD, _DW), jnp.float32),
        mesh=_sc_mesh(),
        scratch_types=[
            pltpu.VMEM((_CH, _DW), jnp.float32),
            pltpu.VMEM((_CPT, _CH), jnp.int32),
            pltpu.SemaphoreType.DMA,
            pltpu.VMEM_SHARED((_NPAD, _DW), jnp.float32),
        ],
    )


def _agg_body(table, srcb, dstb, zeros, out, idx_s, idx_d, rows, sem, acc):
    c = lax.axis_index("c")
    s = lax.axis_index("s")
    base = lax.select(c == 0, s * _CPT_A, _NS * _CPT_A + s * _CPT_B)
    cnt = lax.select(c == 0, _CPT_A, _CPT_B)
    pltpu.sync_copy(zeros, acc.at[pl.ds(s * _RPT, _RPT)])
    pltpu.sync_copy(srcb.at[pl.ds(base, _CPT_MAX)], idx_s)
    pltpu.sync_copy(dstb.at[pl.ds(base, _CPT_MAX)], idx_d)
    plsc.subcore_barrier()

    def chunk(j, carry):
        @pl.when(j < cnt)
        def _():
            pltpu.async_copy(table.at[idx_s.at[j]], rows, sem).wait()
            pltpu.sync_copy(rows, acc.at[idx_d.at[j]], add=True)
        return carry

    lax.fori_loop(0, _CPT_MAX, chunk, 0)
    plsc.subcore_barrier()
    pltpu.sync_copy(acc.at[pl.ds(s * _RPT, _RPT)],
                    out.at[c, pl.ds(s * _RPT, _RPT)])


@functools.lru_cache(maxsize=None)
def _agg_call():
    return pl.kernel(
        _agg_body,
        out_type=jax.ShapeDtypeStruct((_NC, _NPAD, _CIN), jnp.float32),
        mesh=_sc_mesh(),
        scratch_types=[
            pltpu.VMEM((_CPT_MAX, _CH), jnp.int32),
            pltpu.VMEM((_CPT_MAX, _CH), jnp.int32),
            pltpu.VMEM((_CH, _CIN), jnp.float32),
            pltpu.SemaphoreType.DMA,
            pltpu.VMEM_SHARED((_NPAD, _CIN), jnp.float32),
        ],
    )


def _prep_body(x_ref, d0_ref, d1_ref, xs_ref, dinv_ref):
    deg = d0_ref[:, 0:1] + d1_ref[:, 0:1] + 1.0
    dinv = lax.rsqrt(deg)
    dinv_ref[...] = dinv
    xs_ref[...] = x_ref[...] * dinv


def _layer1_body(a0_ref, a1_ref, xs_ref, dinv_ref, w_ref, b_ref,
                 ha_ref, hb_ref):
    dinv = dinv_ref[...]
    y = (a0_ref[...] + a1_ref[...] + xs_ref[...]) * dinv
    h = jnp.maximum(
        jnp.dot(y, w_ref[...], preferred_element_type=jnp.float32)
        + b_ref[...], 0.0) * dinv
    ha_ref[...] = h[:, :_CIN]
    hb_ref[...] = h[:, _CIN:]


def _layer2_pool_body(gl_ref, a2a0, a2a1, a2b0, a2b1, hsa, hsb,
                      dinv_ref, bat_ref, w_ref, b_ref, pool_ref):
    blk = pl.program_id(0)

    @pl.when(blk == 0)
    def _init():
        pool_ref[...] = jnp.full_like(pool_ref[...], -jnp.inf)

    dinv = dinv_ref[...]
    ya = (a2a0[...] + a2a1[...] + hsa[...]) * dinv
    yb = (a2b0[...] + a2b1[...] + hsb[...]) * dinv
    y = jnp.concatenate([ya, yb], axis=1)
    h = jnp.maximum(
        jnp.dot(y, w_ref[...], preferred_element_type=jnp.float32)
        + b_ref[...], 0.0)
    bat = bat_ref[...]

    def upd(g, carry):
        m = bat == g
        mx = jnp.max(jnp.where(m, h, -jnp.inf), axis=0, keepdims=True)
        pool_ref[pl.ds(g, 1), :] = jnp.maximum(pool_ref[pl.ds(g, 1), :], mx)
        return carry

    lax.fori_loop(gl_ref[0, blk], gl_ref[1, blk] + 1, upd, 0)


def _head_body(p_ref, m1w, m1b, g1, be1, m2w, m2b, g2, be2, lw, lb, o_ref):
    p = p_ref[...]
    p = jnp.where(jnp.isfinite(p), p, 0.0)
    eps = 1e-5

    def bn(z, g, b):
        m = jnp.mean(z, axis=0, keepdims=True)
        v = jnp.mean((z - m) ** 2, axis=0, keepdims=True)
        return g[...] * (z - m) / jnp.sqrt(v + eps) + b[...]

    z = jnp.maximum(jnp.dot(p, m1w[...], preferred_element_type=jnp.float32)
                    + m1b[...], 0.0)
    z = bn(z, g1, be1)
    z = jnp.maximum(jnp.dot(z, m2w[...], preferred_element_type=jnp.float32)
                    + m2b[...], 0.0)
    z = bn(z, g2, be2)
    z = jnp.dot(z, lw[...], preferred_element_type=jnp.float32) + lb[...]
    zmax = jnp.max(z, axis=1, keepdims=True)
    zs = z - zmax
    o_ref[...] = zs - jnp.log(jnp.sum(jnp.exp(zs), axis=1, keepdims=True))


def _row_spec(rb, cols):
    return pl.BlockSpec((rb, cols), lambda i, *_: (i, 0))


def _full_spec(shape):
    return pl.BlockSpec(shape, lambda i, *_: tuple(0 for _ in shape))


def kernel(x, edge_index, batch, W1, b1, W2, b2, M1W, M1b, BN1g, BN1b,
           M2W, M2b, BN2g, BN2b, LW, Lb):
    f32 = jnp.float32
    src = edge_index[0].astype(jnp.int32)
    dst = edge_index[1].astype(jnp.int32)
    pad = _EROWS * _CH - _E
    srcb = jnp.concatenate([src, jnp.zeros((pad,), jnp.int32)])
    srcb = srcb.reshape(_EROWS, _CH)
    dstb = jnp.concatenate([dst, jnp.full((pad,), _N, jnp.int32)])
    dstb = dstb.reshape(_EROWS, _CH)

    zeros_deg = jnp.zeros((_RPT, _DW), f32)
    ones_deg = jnp.ones((_CH, _DW), f32)
    zeros_agg = zeros_deg

    degp = _deg_call()(dstb, zeros_deg, ones_deg)[:, :_N]

    rb = 1000
    grid = (_N // rb,)
    xs, dinv = pl.pallas_call(
        _prep_body,
        grid=grid,
        in_specs=[_row_spec(rb, _CIN), _row_spec(rb, _DW), _row_spec(rb, _DW)],
        out_specs=[_row_spec(rb, _CIN), _row_spec(rb, 1)],
        out_shape=[jax.ShapeDtypeStruct((_N, _CIN), f32),
                   jax.ShapeDtypeStruct((_N, 1), f32)],
    )(x, degp[0], degp[1])

    a1 = _agg_call()(xs, srcb, dstb, zeros_agg)[:, :_N]

    rd = 400
    gridd = (_N // rd,)
    hsa, hsb = pl.pallas_call(
        _layer1_body,
        grid=gridd,
        in_specs=[_row_spec(rd, _CIN), _row_spec(rd, _CIN),
                  _row_spec(rd, _CIN), _row_spec(rd, 1),
                  _full_spec((_CIN, 2 * _CIN)), _full_spec((1, 2 * _CIN))],
        out_specs=[_row_spec(rd, _CIN), _row_spec(rd, _CIN)],
        out_shape=[jax.ShapeDtypeStruct((_N, _CIN), f32),
                   jax.ShapeDtypeStruct((_N, _CIN), f32)],
    )(a1[0], a1[1], xs, dinv, W1, b1.reshape(1, -1))

    a2a = _agg_call()(hsa, srcb, dstb, zeros_agg)[:, :_N]
    a2b = _agg_call()(hsb, srcb, dstb, zeros_agg)[:, :_N]

    nblk = _N // rd
    bi = jnp.arange(nblk, dtype=jnp.int32)
    bat32 = batch.astype(jnp.int32)
    gl = jnp.stack([bat32[bi * rd], bat32[(bi + 1) * rd - 1]])
    pooled = pl.pallas_call(
        _layer2_pool_body,
        grid_spec=pltpu.PrefetchScalarGridSpec(
            num_scalar_prefetch=1,
            grid=gridd,
            in_specs=[_row_spec(rd, _CIN), _row_spec(rd, _CIN),
                      _row_spec(rd, _CIN), _row_spec(rd, _CIN),
                      _row_spec(rd, _CIN), _row_spec(rd, _CIN),
                      _row_spec(rd, 1), _row_spec(rd, 1),
                      _full_spec((2 * _CIN, 4 * _CIN)),
                      _full_spec((1, 4 * _CIN))],
            out_specs=_full_spec((_G, 4 * _CIN)),
        ),
        out_shape=jax.ShapeDtypeStruct((_G, 4 * _CIN), f32),
    )(gl, a2a[0], a2a[1], a2b[0], a2b[1], hsa, hsb, dinv,
      bat32.reshape(_N, 1), W2, b2.reshape(1, -1))

    out = pl.pallas_call(
        _head_body,
        grid=(1,),
        in_specs=[_full_spec((_G, 4 * _CIN)),
                  _full_spec((4 * _CIN, 32)), _full_spec((1, 32)),
                  _full_spec((1, 32)), _full_spec((1, 32)),
                  _full_spec((32, 64)), _full_spec((1, 64)),
                  _full_spec((1, 64)), _full_spec((1, 64)),
                  _full_spec((64, 40)), _full_spec((1, 40))],
        out_specs=_full_spec((_G, 40)),
        out_shape=jax.ShapeDtypeStruct((_G, 40), f32),
    )(pooled, M1W, M1b.reshape(1, -1), BN1g.reshape(1, -1),
      BN1b.reshape(1, -1), M2W, M2b.reshape(1, -1), BN2g.reshape(1, -1),
      BN2b.reshape(1, -1), LW, Lb.reshape(1, -1))
    return out
